# Initial kernel scaffold; baseline (speedup 1.0000x reference)
import jax
import jax.numpy as jnp
from jax.experimental import pallas as pl


def kernel(x, gate_weight, expert_w, expert_b):
    raise NotImplementedError("write your pallas kernel here")


# speedup vs baseline: 1.5627x; 1.5627x over previous
import functools

import jax
import jax.numpy as jnp
from jax.experimental import pallas as pl
from jax.experimental.pallas import tpu as pltpu

E = 64
D = 768
CAPACITY = 2


def _gates_kernel(x_ref, gw_ref, gates_ref, *, k_total):
    logits = jnp.dot(x_ref[...], gw_ref[...].T, preferred_element_type=jnp.float32)
    m = jnp.max(logits, axis=-1, keepdims=True)
    ex = jnp.exp(logits - m)
    scores = ex / jnp.sum(ex, axis=-1, keepdims=True)

    bits = jax.lax.bitcast_convert_type(scores, jnp.int32)

    def body(_, lohi):
        lo, hi = lohi
        mid = jax.lax.div(lo + hi + 1, jnp.int32(2))
        cnt = jnp.sum((bits >= mid).astype(jnp.int32))
        take = cnt >= k_total
        return jnp.where(take, mid, lo), jnp.where(take, hi, mid - 1)

    lo, _ = jax.lax.fori_loop(
        0, 30, body, (jnp.int32(0), jnp.int32(0x3F800000))
    )
    gates_ref[...] = jnp.where(bits >= lo, scores, 0.0)


def _moe_dense_kernel(x_ref, gates_ref, w_ref, b_ref, o_ref):
    e = pl.program_id(1)
    g_all = gates_ref[...]
    col = jax.lax.broadcasted_iota(jnp.int32, (1, E), 1)
    g = jnp.sum(jnp.where(col == e, g_all, 0.0), axis=1)
    y = jnp.dot(x_ref[...], w_ref[0].T, preferred_element_type=jnp.float32)
    y = (y + b_ref[0]) * g[:, None]

    @pl.when(e == 0)
    def _init():
        o_ref[...] = y

    @pl.when(e > 0)
    def _acc():
        o_ref[...] += y


def kernel(x, gate_weight, expert_w, expert_b):
    Bb, s, _ = x.shape
    x_flat = x.reshape(-1, D)
    S = x_flat.shape[0]
    k_total = S * CAPACITY

    gates = pl.pallas_call(
        functools.partial(_gates_kernel, k_total=k_total),
        out_shape=jax.ShapeDtypeStruct((S, E), jnp.float32),
    )(x_flat, gate_weight)

    BS = 512
    out = pl.pallas_call(
        _moe_dense_kernel,
        grid=(S // BS, E),
        in_specs=[
            pl.BlockSpec((BS, D), lambda i, e: (i, 0)),
            pl.BlockSpec((BS, E), lambda i, e: (i, 0)),
            pl.BlockSpec((1, D, D), lambda i, e: (e, 0, 0)),
            pl.BlockSpec((1, 1, D), lambda i, e: (e, 0, 0)),
        ],
        out_specs=pl.BlockSpec((BS, D), lambda i, e: (i, 0)),
        out_shape=jax.ShapeDtypeStruct((S, D), jnp.float32),
        compiler_params=pltpu.CompilerParams(
            dimension_semantics=("parallel", "arbitrary"),
        ),
    )(x_flat, gates, expert_w, expert_b.reshape(E, 1, D))

    return out.reshape(Bb, s, D)


# speedup vs baseline: 1.8307x; 1.1715x over previous
import functools

import jax
import jax.numpy as jnp
from jax import lax
from jax.experimental import pallas as pl
from jax.experimental.pallas import tpu as pltpu
from jax.experimental.pallas import tpu_sc as plsc

E = 64
D = 768
CAPACITY = 2
S = 8192
K_TOTAL = S * CAPACITY
TILE = 128
NT = 192
PADN = NT * TILE
NW = 32
L = 16


def _read_lane_i32(vecref, idx):
    base = pl.multiple_of(lax.div(idx, L) * L, L)
    win = vecref[pl.ds(base, L)]
    lane = idx - base
    return jnp.sum(jnp.where(lax.iota(jnp.int32, L) == lane, win, 0))


def _gates_kernel(x_ref, gw_ref, gt_ref, offp_ref, te_ref):
    logits = lax.dot_general(
        gw_ref[...], x_ref[...], (((1,), (1,)), ((), ())),
        preferred_element_type=jnp.float32)
    m = jnp.max(logits, axis=0, keepdims=True)
    ex = jnp.exp(logits - m)
    scores = ex / jnp.sum(ex, axis=0, keepdims=True)

    bits = lax.bitcast_convert_type(scores, jnp.int32)

    def body(_, lohi):
        lo, hi = lohi
        mid = lax.div(lo + hi + 1, jnp.int32(2))
        cnt = jnp.sum((bits >= mid).astype(jnp.int32))
        take = cnt >= K_TOTAL
        return jnp.where(take, mid, lo), jnp.where(take, hi, mid - 1)

    lo, _ = lax.fori_loop(0, 30, body, (jnp.int32(0), jnp.int32(0x3F800000)))
    sel = bits >= lo
    gt_ref[...] = jnp.where(sel, scores, 0.0)

    ones_row = jnp.ones((1, S), jnp.float32)
    cnt_row = lax.dot_general(
        ones_row, sel.astype(jnp.float32), (((1,), (1,)), ((), ())),
        preferred_element_type=jnp.float32)
    padded = jnp.floor((cnt_row + (TILE - 1)) / TILE) * TILE
    r = lax.broadcasted_iota(jnp.int32, (E, E), 0)
    c = lax.broadcasted_iota(jnp.int32, (E, E), 1)
    upper = (r <= c).astype(jnp.float32)
    incl = jnp.dot(padded, upper, preferred_element_type=jnp.float32)
    incl = jnp.minimum(incl, float(PADN))
    excl = jnp.minimum(incl - padded, float(PADN))
    total = jnp.max(incl)

    lane = lax.broadcasted_iota(jnp.int32, (1, 2 * E), 1)
    both = jnp.concatenate([excl, incl], axis=1)
    offp_ref[...] = jnp.where(lane < E, both, total).astype(jnp.int32)

    tv = lax.broadcasted_iota(jnp.int32, (256, 1), 0).astype(jnp.float32) * TILE
    tef = jnp.sum((incl <= tv).astype(jnp.float32), axis=1, keepdims=True)
    te_ref[...] = jnp.minimum(tef, float(E - 1)).astype(jnp.int32)


def _dispatch_kernel(gt_hbm, offp_hbm, tok_hbm, gate_hbm,
                     row_v, tokbuf, gatebuf, offp_v, cntbuf, zt_v, zg_v):
    wid = lax.axis_index("s") * 2 + lax.axis_index("c")
    pltpu.sync_copy(offp_hbm, offp_v)

    zero16i = jnp.zeros((L,), jnp.int32)
    zero16f = jnp.zeros((L,), jnp.float32)
    iota16 = lax.iota(jnp.int32, L)
    for zi in range(TILE // L):
        zt_v[pl.ds(zi * L, L)] = zero16i
        zg_v[pl.ds(zi * L, L)] = zero16f

    def do_expert(e):
        pltpu.sync_copy(gt_hbm.at[e], row_v)

        def zbody(i, _):
            o = pl.multiple_of(i * L, L)
            tokbuf[pl.ds(o, L)] = zero16i
            gatebuf[pl.ds(o, L)] = zero16f
            return 0
        lax.fori_loop(0, (S + TILE) // L, zbody, 0)

        def sbody(i, cnt):
            v = row_v[pl.ds(pl.multiple_of(i * L, L), L)]
            msk = v > 0.0
            pc = plsc.cumsum(msk.astype(jnp.int32))
            pos = cnt + pc - 1
            plsc.store_scatter(tokbuf, [pos], iota16 + i * L, mask=msk)
            plsc.store_scatter(gatebuf, [pos], v, mask=msk)
            return cnt + plsc.all_reduce_population_count(msk)
        cnt = lax.fori_loop(0, S // L, sbody, zero16i)
        cntbuf[...] = cnt
        c = cntbuf[...][0]

        excl = _read_lane_i32(offp_v, e)
        nch = jnp.minimum(lax.div(c + TILE - 1, TILE),
                          lax.div(PADN - excl, TILE))

        def wbody(j, _):
            jo = pl.multiple_of(j * TILE, TILE)
            go = pl.multiple_of(excl + j * TILE, TILE)
            pltpu.sync_copy(tokbuf.at[pl.ds(jo, TILE)],
                            tok_hbm.at[pl.ds(go, TILE)])
            pltpu.sync_copy(gatebuf.at[pl.ds(jo, TILE)],
                            gate_hbm.at[pl.ds(go, TILE)])
            return 0
        lax.fori_loop(0, nch, wbody, 0)

    do_expert(wid * 2)
    do_expert(wid * 2 + 1)

    @pl.when(wid == NW - 1)
    def _tail():
        total = offp_v[pl.ds(E, L)][0]

        def tbody(j, _):
            off = pl.multiple_of(total + j * TILE, TILE)

            @pl.when(off < PADN)
            def _():
                pltpu.sync_copy(zt_v, tok_hbm.at[pl.ds(off, TILE)])
                pltpu.sync_copy(zg_v, gate_hbm.at[pl.ds(off, TILE)])
            return 0
        lax.fori_loop(0, NT, tbody, 0)


def _gather_kernel(tok_hbm, x_hbm, xg_hbm, idx_v, rows_v, sem):
    wid = lax.axis_index("s") * 2 + lax.axis_index("c")
    per_w = PADN // NW
    for j in range(per_w // TILE):
        base = pl.multiple_of(wid * per_w + j * TILE, TILE)
        pltpu.sync_copy(tok_hbm.at[pl.ds(base, TILE)], idx_v)
        pltpu.async_copy(x_hbm.at[idx_v], rows_v, sem).wait()
        pltpu.sync_copy(rows_v, xg_hbm.at[pl.ds(base, TILE)])


def _gmm_kernel(te_ref, xg_ref, w_ref, b_ref, gl_ref, yg_ref):
    y = lax.dot_general(
        xg_ref[...], w_ref[0], (((1,), (1,)), ((), ())),
        preferred_element_type=jnp.float32)
    g = gl_ref[0, 0]
    yg_ref[...] = (y + b_ref[0]) * g[:, None]


def _combine_kernel(yg_hbm, tok_hbm, gate_hbm, out_hbm,
                    acc, tokch, gch, msrc, mloc, cntbuf, rows_v, sem):
    wid = lax.axis_index("s") * 2 + lax.axis_index("c")
    TB = 64
    CH = 2048
    zero16i = jnp.zeros((L,), jnp.int32)
    iota16 = lax.iota(jnp.int32, L)
    MCAP = 4096

    for p in range(S // (NW * TB)):
        tb = wid * (S // NW) + p * TB

        def zacc(i, _):
            acc[pl.ds(pl.multiple_of(i * L, L), L)] = jnp.zeros((L,), jnp.float32)
            return 0
        lax.fori_loop(0, (TB + 1) * D // L, zacc, 0)

        def zm(i, _):
            for q in range(TB // L):
                msrc[i, pl.ds(q * L, L)] = zero16i
            o = pl.multiple_of(i * TB, TB)
            for q in range(TB // L):
                mloc[pl.ds(o + q * L, L)] = jnp.full((L,), TB, jnp.int32)
            return 0
        lax.fori_loop(0, MCAP // TB, zm, 0)

        def scan_chunk(cidx, cnt):
            cb = pl.multiple_of(cidx * CH, CH)
            pltpu.sync_copy(tok_hbm.at[pl.ds(cb, CH)], tokch)
            pltpu.sync_copy(gate_hbm.at[pl.ds(cb, CH)], gch)

            def sbody(i, cnt):
                o = pl.multiple_of(i * L, L)
                tv = tokch[pl.ds(o, L)]
                gv = gch[pl.ds(o, L)]
                msk = (tv >= tb) & (tv < tb + TB) & (gv > 0.0)
                pc = plsc.cumsum(msk.astype(jnp.int32))
                pos = cnt + pc - 1
                plsc.store_scatter(msrc, [lax.div(pos, TB), lax.rem(pos, TB)],
                                   cb + i * L + iota16, mask=msk)
                plsc.store_scatter(mloc, [pos], tv - tb, mask=msk)
                return cnt + plsc.all_reduce_population_count(msk)
            return lax.fori_loop(0, CH // L, sbody, cnt)

        cnt = lax.fori_loop(0, PADN // CH, scan_chunk, zero16i)
        cntbuf[...] = cnt
        c = cntbuf[...][0]
        nb = lax.div(c + TB - 1, TB)

        def bbody(b, _):
            pltpu.async_copy(yg_hbm.at[msrc.at[b]], rows_v, sem).wait()
            nrows = jnp.minimum(c - b * TB, TB)

            def rbody(j, _):
                lj = _read_lane_i32(mloc, b * TB + j)
                for i in range(D // L):
                    sl = pl.ds(pl.multiple_of(lj * D + i * L, L), L)
                    acc[sl] = acc[sl] + rows_v[j, pl.ds(i * L, L)]
                return 0
            lax.fori_loop(0, nrows, rbody, 0)
            return 0
        lax.fori_loop(0, nb, bbody, 0)

        pltpu.sync_copy(acc.at[pl.ds(0, TB * D)],
                        out_hbm.at[pl.ds(pl.multiple_of(tb * D, TB * D), TB * D)])


def kernel(x, gate_weight, expert_w, expert_b):
    Bb, s, _ = x.shape
    x_flat = x.reshape(-1, D)

    gatesT, offp, te = pl.pallas_call(
        _gates_kernel,
        out_shape=(
            jax.ShapeDtypeStruct((E, S), jnp.float32),
            jax.ShapeDtypeStruct((1, 2 * E), jnp.int32),
            jax.ShapeDtypeStruct((256, 1), jnp.int32),
        ),
    )(x_flat, gate_weight)

    mesh = plsc.VectorSubcoreMesh(core_axis_name="c", subcore_axis_name="s")

    tok, gate = pl.kernel(
        _dispatch_kernel,
        out_type=(
            jax.ShapeDtypeStruct((PADN,), jnp.int32),
            jax.ShapeDtypeStruct((PADN,), jnp.float32),
        ),
        mesh=mesh,
        compiler_params=pltpu.CompilerParams(needs_layout_passes=False),
        scratch_types=[
            pltpu.VMEM((S,), jnp.float32),
            pltpu.VMEM((S + TILE,), jnp.int32),
            pltpu.VMEM((S + TILE,), jnp.float32),
            pltpu.VMEM((2 * E,), jnp.int32),
            pltpu.VMEM((L,), jnp.int32),
            pltpu.VMEM((TILE,), jnp.int32),
            pltpu.VMEM((TILE,), jnp.float32),
        ],
    )(gatesT, offp.reshape(2 * E))

    xg = pl.kernel(
        _gather_kernel,
        out_type=jax.ShapeDtypeStruct((PADN, D), jnp.float32),
        mesh=mesh,
        compiler_params=pltpu.CompilerParams(needs_layout_passes=False),
        scratch_types=[
            pltpu.VMEM((TILE,), jnp.int32),
            pltpu.VMEM((TILE, D), jnp.float32),
            pltpu.SemaphoreType.DMA,
        ],
    )(tok, x_flat)

    yg = pl.pallas_call(
        _gmm_kernel,
        grid_spec=pltpu.PrefetchScalarGridSpec(
            num_scalar_prefetch=1,
            grid=(NT,),
            in_specs=[
                pl.BlockSpec((TILE, D), lambda i, te_r: (i, 0)),
                pl.BlockSpec((1, D, D), lambda i, te_r: (te_r[i], 0, 0)),
                pl.BlockSpec((1, 1, D), lambda i, te_r: (te_r[i], 0, 0)),
                pl.BlockSpec((1, 1, TILE), lambda i, te_r: (i, 0, 0)),
            ],
            out_specs=pl.BlockSpec((TILE, D), lambda i, te_r: (i, 0)),
        ),
        out_shape=jax.ShapeDtypeStruct((PADN, D), jnp.float32),
        compiler_params=pltpu.CompilerParams(
            dimension_semantics=("arbitrary",),
        ),
    )(te.reshape(256), xg, expert_w, expert_b.reshape(E, 1, D),
      gate.reshape(NT, 1, TILE))

    out = pl.kernel(
        _combine_kernel,
        out_type=jax.ShapeDtypeStruct((S * D,), jnp.float32),
        mesh=mesh,
        compiler_params=pltpu.CompilerParams(needs_layout_passes=False),
        scratch_types=[
            pltpu.VMEM(((64 + 1) * D,), jnp.float32),
            pltpu.VMEM((2048,), jnp.int32),
            pltpu.VMEM((2048,), jnp.float32),
            pltpu.VMEM((64, 64), jnp.int32),
            pltpu.VMEM((4096 + L,), jnp.int32),
            pltpu.VMEM((L,), jnp.int32),
            pltpu.VMEM((64, D), jnp.float32),
            pltpu.SemaphoreType.DMA,
        ],
    )(yg, tok, gate)

    return out.reshape(Bb, s, D)


# speedup vs baseline: 2.7214x; 1.4865x over previous
import functools

import jax
import jax.numpy as jnp
from jax import lax
from jax.experimental import pallas as pl
from jax.experimental.pallas import tpu as pltpu
from jax.experimental.pallas import tpu_sc as plsc

E = 64
D = 768
CAPACITY = 2
S = 8192
K_TOTAL = S * CAPACITY
TILE = 128
NT = 192
PADN = NT * TILE
NW = 32
L = 16
GC = 64
CHT = 1024
SLICE = PADN // 16


def _read_lane_i32(vecref, idx):
    base = pl.multiple_of(lax.div(idx, L) * L, L)
    win = vecref[pl.ds(base, L)]
    lane = idx - base
    return jnp.sum(jnp.where(lax.iota(jnp.int32, L) == lane, win, 0))


def _gates_kernel(x_ref, gw_ref, gt_ref, offp_ref, te_ref):
    logits = lax.dot_general(
        gw_ref[...], x_ref[...], (((1,), (1,)), ((), ())),
        preferred_element_type=jnp.float32)
    m = jnp.max(logits, axis=0, keepdims=True)
    ex = jnp.exp(logits - m)
    scores = ex / jnp.sum(ex, axis=0, keepdims=True)

    bits = lax.bitcast_convert_type(scores, jnp.int32)

    def body(_, lohi):
        lo, hi = lohi
        mid = lax.div(lo + hi + 1, jnp.int32(2))
        cnt = jnp.sum((bits >= mid).astype(jnp.int32))
        take = cnt >= K_TOTAL
        return jnp.where(take, mid, lo), jnp.where(take, hi, mid - 1)

    lo, _ = lax.fori_loop(0, 30, body, (jnp.int32(0), jnp.int32(0x3F800000)))
    sel = bits >= lo
    gt_ref[...] = jnp.where(sel, scores, 0.0)

    ones_row = jnp.ones((1, S), jnp.float32)
    cnt_row = lax.dot_general(
        ones_row, sel.astype(jnp.float32), (((1,), (1,)), ((), ())),
        preferred_element_type=jnp.float32)
    padded = jnp.floor((cnt_row + (TILE - 1)) / TILE) * TILE
    r = lax.broadcasted_iota(jnp.int32, (E, E), 0)
    c = lax.broadcasted_iota(jnp.int32, (E, E), 1)
    upper = (r <= c).astype(jnp.float32)
    incl = jnp.dot(padded, upper, preferred_element_type=jnp.float32)
    incl = jnp.minimum(incl, float(PADN))
    excl = jnp.minimum(incl - padded, float(PADN))
    total = jnp.max(incl)

    lane = lax.broadcasted_iota(jnp.int32, (1, 2 * E), 1)
    both = jnp.concatenate([excl, incl], axis=1)
    offp_ref[...] = jnp.where(lane < E, both, total).astype(jnp.int32)

    tv = lax.broadcasted_iota(jnp.int32, (256, 1), 0).astype(jnp.float32) * TILE
    tef = jnp.sum((incl <= tv).astype(jnp.float32), axis=1, keepdims=True)
    te_ref[...] = jnp.minimum(tef, float(E - 1)).astype(jnp.int32)


def _dispatch_kernel(gt_hbm, offp_hbm, x_hbm, tok_hbm, gate_hbm, xg_hbm,
                     row_v, tokbuf, gatebuf, offp_v, cntbuf, zt_v, zg_v,
                     rows_a, rows_b, sem_a, sem_b):
    wid = lax.axis_index("s") * 2 + lax.axis_index("c")
    pltpu.sync_copy(offp_hbm, offp_v)

    zero16i = jnp.zeros((L,), jnp.int32)
    zero16f = jnp.zeros((L,), jnp.float32)
    sent16 = jnp.full((L,), S, jnp.int32)
    iota16 = lax.iota(jnp.int32, L)
    for zi in range(TILE // L):
        zt_v[pl.ds(zi * L, L)] = sent16
        zg_v[pl.ds(zi * L, L)] = zero16f

    def do_expert(e):
        pltpu.sync_copy(gt_hbm.at[e], row_v)

        def zbody(i, _):
            o = pl.multiple_of(i * L, L)
            tokbuf[pl.ds(o, L)] = sent16
            gatebuf[pl.ds(o, L)] = zero16f
            return 0
        lax.fori_loop(0, (S + TILE) // L, zbody, 0)

        def sbody(i, cnt):
            v = row_v[pl.ds(pl.multiple_of(i * L, L), L)]
            msk = v > 0.0
            pc = plsc.cumsum(msk.astype(jnp.int32))
            pos = cnt + pc - 1
            plsc.store_scatter(tokbuf, [pos], iota16 + i * L, mask=msk)
            plsc.store_scatter(gatebuf, [pos], v, mask=msk)
            return cnt + pc[L - 1]
        cnt = lax.fori_loop(0, S // L, sbody, zero16i)
        cntbuf[...] = cnt
        c = cntbuf[...][0]

        excl = _read_lane_i32(offp_v, e)
        nch = jnp.minimum(lax.div(c + TILE - 1, TILE),
                          lax.div(PADN - excl, TILE))

        def wbody(j, _):
            jo = pl.multiple_of(j * TILE, TILE)
            go = pl.multiple_of(excl + j * TILE, TILE)
            pltpu.sync_copy(tokbuf.at[pl.ds(jo, TILE)],
                            tok_hbm.at[pl.ds(go, TILE)])
            pltpu.sync_copy(gatebuf.at[pl.ds(jo, TILE)],
                            gate_hbm.at[pl.ds(go, TILE)])
            return 0
        lax.fori_loop(0, nch, wbody, 0)

        ngc = nch * (TILE // GC)

        def gbody(j2, _):
            j = j2 * 2
            c0 = j < ngc
            c1 = j + 1 < ngc
            jo0 = pl.multiple_of(j * GC, GC)
            jo1 = pl.multiple_of(j * GC + GC, GC)
            go0 = pl.multiple_of(excl + j * GC, GC)
            go1 = pl.multiple_of(excl + j * GC + GC, GC)

            @pl.when(c0)
            def _():
                pltpu.async_copy(x_hbm.at[tokbuf.at[pl.ds(jo0, GC)]],
                                 rows_a, sem_a)

            @pl.when(c1)
            def _():
                pltpu.async_copy(x_hbm.at[tokbuf.at[pl.ds(jo1, GC)]],
                                 rows_b, sem_b)

            @pl.when(c0)
            def _():
                pltpu.make_async_copy(x_hbm.at[tokbuf.at[pl.ds(jo0, GC)]],
                                      rows_a, sem_a).wait()
                pltpu.sync_copy(rows_a, xg_hbm.at[pl.ds(go0, GC)])

            @pl.when(c1)
            def _():
                pltpu.make_async_copy(x_hbm.at[tokbuf.at[pl.ds(jo1, GC)]],
                                      rows_b, sem_b).wait()
                pltpu.sync_copy(rows_b, xg_hbm.at[pl.ds(go1, GC)])
            return 0
        lax.fori_loop(0, lax.div(ngc + 1, 2), gbody, 0)

    do_expert(wid * 2)
    do_expert(wid * 2 + 1)

    @pl.when(wid == NW - 1)
    def _tail():
        total = offp_v[pl.ds(E, L)][0]

        def tbody(j, _):
            off = pl.multiple_of(total + j * TILE, TILE)

            @pl.when(off < PADN)
            def _():
                pltpu.sync_copy(zt_v, tok_hbm.at[pl.ds(off, TILE)])
                pltpu.sync_copy(zg_v, gate_hbm.at[pl.ds(off, TILE)])
            return 0
        lax.fori_loop(0, NT, tbody, 0)


def _gmm_kernel(te_ref, xg_ref, w_ref, b_ref, gl_ref, yg_ref):
    y = lax.dot_general(
        xg_ref[...], w_ref[0], (((1,), (1,)), ((), ())),
        preferred_element_type=jnp.float32)
    g = gl_ref[0, 0]
    gc = g[:, None]
    yg_ref[...] = jnp.where(gc > 0.0, (y + b_ref[0]) * gc, 0.0)


L1CAP = 8192
L2CAP = 2048
TB = 32
NP = (S // NW) // TB


def _combine_kernel(yg_hbm, tok_hbm, out_hbm,
                    tokch, l1src, l1loc, l2src, l2loc, cntbuf,
                    acc, rows_a, rows_b, sem_a, sem_b):
    wid = lax.axis_index("s") * 2 + lax.axis_index("c")
    mybase = wid * (S // NW)
    zero16i = jnp.zeros((L,), jnp.int32)
    iota16 = lax.iota(jnp.int32, L)

    pltpu.sync_copy(tok_hbm, tokch)

    def scan1(i, cnt):
        o = pl.multiple_of(i * L, L)
        tv = tokch[pl.ds(o, L)]
        msk = (tv >= mybase) & (tv < mybase + S // NW)
        pc = plsc.cumsum(msk.astype(jnp.int32))
        pos = cnt + pc - 1
        msk = msk & (pos < L1CAP)
        plsc.store_scatter(l1src, [pos], i * L + iota16, mask=msk)
        plsc.store_scatter(l1loc, [pos], tv - mybase, mask=msk)
        return cnt + pc[L - 1]
    cnt1 = lax.fori_loop(0, PADN // L, scan1, zero16i)
    cntbuf[...] = jnp.minimum(cnt1, L1CAP)
    c1 = cntbuf[...][0]
    n1 = lax.div(c1 + L - 1, L)

    for p in range(NP):
        def zacc(i, _):
            for q in range(D // L):
                acc[i, pl.ds(q * L, L)] = jnp.zeros((L,), jnp.float32)
            return 0
        lax.fori_loop(0, TB, zacc, 0)

        def zl2(i, _):
            for q in range(TB // L):
                l2src[i, pl.ds(q * L, L)] = zero16i
            return 0
        lax.fori_loop(0, L2CAP // TB, zl2, 0)

        lo = p * TB

        def scan2(i, cnt):
            o = pl.multiple_of(i * L, L)
            lv = l1loc[pl.ds(o, L)]
            sv = l1src[pl.ds(o, L)]
            valid = (i * L + iota16) < c1
            msk = valid & (lv >= lo) & (lv < lo + TB)
            pc = plsc.cumsum(msk.astype(jnp.int32))
            pos = cnt + pc - 1
            msk = msk & (pos < L2CAP)
            plsc.store_scatter(l2src, [lax.div(pos, TB), lax.rem(pos, TB)],
                               sv, mask=msk)
            plsc.store_scatter(l2loc, [pos], lv - lo, mask=msk)
            return cnt + pc[L - 1]
        cnt2 = lax.fori_loop(0, n1, scan2, zero16i)
        cntbuf[...] = jnp.minimum(cnt2, L2CAP)
        c2 = cntbuf[...][0]
        nb = lax.div(c2 + TB - 1, TB)

        def addrows(rbuf, b):
            nr = jnp.minimum(c2 - b * TB, TB)

            def rbody(j, _):
                lj = _read_lane_i32(l2loc, b * TB + j)
                for q in range(D // L):
                    sl = pl.ds(pl.multiple_of(lj * 0 + q * L, L), L)
                    acc[lj, sl] = acc[lj, sl] + rbuf[j, pl.ds(q * L, L)]
                return 0
            lax.fori_loop(0, nr, rbody, 0)

        def bbody(b2, _):
            b = b2 * 2
            c0 = b < nb
            c1b = b + 1 < nb

            @pl.when(c0)
            def _():
                pltpu.async_copy(yg_hbm.at[l2src.at[b]], rows_a, sem_a)

            @pl.when(c1b)
            def _():
                pltpu.async_copy(yg_hbm.at[l2src.at[b + 1]], rows_b, sem_b)

            @pl.when(c0)
            def _():
                pltpu.make_async_copy(yg_hbm.at[l2src.at[b]],
                                      rows_a, sem_a).wait()
                addrows(rows_a, b)

            @pl.when(c1b)
            def _():
                pltpu.make_async_copy(yg_hbm.at[l2src.at[b + 1]],
                                      rows_b, sem_b).wait()
                addrows(rows_b, b + 1)
            return 0
        lax.fori_loop(0, lax.div(nb + 1, 2), bbody, 0)

        pltpu.sync_copy(acc, out_hbm.at[pl.ds(mybase + lo, TB)])


def kernel(x, gate_weight, expert_w, expert_b):
    Bb, s, _ = x.shape
    x_flat = x.reshape(-1, D)

    gatesT, offp, te = pl.pallas_call(
        _gates_kernel,
        out_shape=(
            jax.ShapeDtypeStruct((E, S), jnp.float32),
            jax.ShapeDtypeStruct((1, 2 * E), jnp.int32),
            jax.ShapeDtypeStruct((256, 1), jnp.int32),
        ),
    )(x_flat, gate_weight)

    mesh = plsc.VectorSubcoreMesh(core_axis_name="c", subcore_axis_name="s")
    sc_params = pltpu.CompilerParams(needs_layout_passes=False)

    xpad = jnp.concatenate([x_flat, jnp.zeros((8, D), jnp.float32)], axis=0)

    tok, gate, xg = pl.kernel(
        _dispatch_kernel,
        out_type=(
            jax.ShapeDtypeStruct((PADN,), jnp.int32),
            jax.ShapeDtypeStruct((PADN,), jnp.float32),
            jax.ShapeDtypeStruct((PADN, D), jnp.float32),
        ),
        mesh=mesh,
        compiler_params=sc_params,
        scratch_types=[
            pltpu.VMEM((S,), jnp.float32),
            pltpu.VMEM((S + TILE,), jnp.int32),
            pltpu.VMEM((S + TILE,), jnp.float32),
            pltpu.VMEM((2 * E,), jnp.int32),
            pltpu.VMEM((L,), jnp.int32),
            pltpu.VMEM((TILE,), jnp.int32),
            pltpu.VMEM((TILE,), jnp.float32),
            pltpu.VMEM((GC, D), jnp.float32),
            pltpu.VMEM((GC, D), jnp.float32),
            pltpu.SemaphoreType.DMA,
            pltpu.SemaphoreType.DMA,
        ],
    )(gatesT, offp.reshape(2 * E), xpad)

    yg = pl.pallas_call(
        _gmm_kernel,
        grid_spec=pltpu.PrefetchScalarGridSpec(
            num_scalar_prefetch=1,
            grid=(NT,),
            in_specs=[
                pl.BlockSpec((TILE, D), lambda i, te_r: (i, 0)),
                pl.BlockSpec((1, D, D), lambda i, te_r: (te_r[i], 0, 0)),
                pl.BlockSpec((1, 1, D), lambda i, te_r: (te_r[i], 0, 0)),
                pl.BlockSpec((1, 1, TILE), lambda i, te_r: (i, 0, 0)),
            ],
            out_specs=pl.BlockSpec((TILE, D), lambda i, te_r: (i, 0)),
        ),
        out_shape=jax.ShapeDtypeStruct((PADN, D), jnp.float32),
        compiler_params=pltpu.CompilerParams(
            dimension_semantics=("arbitrary",),
        ),
    )(te.reshape(256), xg, expert_w, expert_b.reshape(E, 1, D),
      gate.reshape(NT, 1, TILE))

    out = pl.kernel(
        _combine_kernel,
        out_type=jax.ShapeDtypeStruct((S, D), jnp.float32),
        mesh=mesh,
        compiler_params=sc_params,
        scratch_types=[
            pltpu.VMEM((PADN,), jnp.int32),
            pltpu.VMEM((L1CAP,), jnp.int32),
            pltpu.VMEM((L1CAP,), jnp.int32),
            pltpu.VMEM((L2CAP // TB, TB), jnp.int32),
            pltpu.VMEM((L2CAP + L,), jnp.int32),
            pltpu.VMEM((L,), jnp.int32),
            pltpu.VMEM((TB, D), jnp.float32),
            pltpu.VMEM((TB, D), jnp.float32),
            pltpu.VMEM((TB, D), jnp.float32),
            pltpu.SemaphoreType.DMA,
            pltpu.SemaphoreType.DMA,
        ],
    )(yg, tok)

    return out.reshape(Bb, s, D)
